# Initial kernel scaffold; baseline (speedup 1.0000x reference)
import jax
import jax.numpy as jnp
from jax.experimental import pallas as pl


def kernel(x, W_lin, b_lin, W_eye, b_eye, adj_vals, edge_index, centroids):
    raise NotImplementedError("write your pallas kernel here")


# speedup vs baseline: 4.4391x; 4.4391x over previous
import dataclasses
import functools

import jax
import jax.numpy as jnp
from jax import lax
from jax.experimental import pallas as pl
from jax.experimental.pallas import tpu as pltpu
from jax.experimental.pallas import tpu_sc as plsc

F32 = jnp.float32
I32 = jnp.int32

NC, NS, L = 2, 16, 16
NW = NC * NS


def _sc_compiler_params():
    cp = pltpu.CompilerParams()
    if "needs_layout_passes" in pltpu.CompilerParams.__dataclass_fields__:
        cp = dataclasses.replace(cp, needs_layout_passes=False)
    return cp


def _k1_linear(x, W_lin):
    B, N, CIN = x.shape
    O = W_lin.shape[0]
    RB = 2000

    def body(x_ref, w_ref, z0_ref, z1_ref):
        w = w_ref[...]
        for half, zr in ((0, z0_ref), (1, z1_ref)):
            for j in range(2):
                b = half * 2 + j
                zr[:, j * O:(j + 1) * O] = lax.dot_general(
                    x_ref[b], w, (((1,), (1,)), ((), ())),
                    precision=lax.Precision.HIGHEST,
                    preferred_element_type=F32)

    zshape = jax.ShapeDtypeStruct((N, 2 * O), F32)
    return pl.pallas_call(
        body,
        grid=(N // RB,),
        in_specs=[
            pl.BlockSpec((B, RB, CIN), lambda i: (0, i, 0)),
            pl.BlockSpec((O, CIN), lambda i: (0, 0)),
        ],
        out_specs=[
            pl.BlockSpec((RB, 2 * O), lambda i: (i, 0)),
            pl.BlockSpec((RB, 2 * O), lambda i: (i, 0)),
        ],
        out_shape=(zshape, zshape),
    )(x, W_lin)


def _k2_spmm(z0, z1, rows, cols, vals, Np):
    F = z0.shape[1]
    K = 128
    Ep = rows.shape[0]
    EPS = Ep // NS
    NCHUNK = EPS // K
    RPT = Np // NS
    ZB = 128
    mesh = plsc.VectorSubcoreMesh(core_axis_name="c", subcore_axis_name="s")
    yshape = jax.ShapeDtypeStruct((Np, F), F32)

    @functools.partial(
        pl.kernel,
        out_type=(yshape, yshape),
        mesh=mesh,
        scratch_types=[
            pltpu.VMEM_SHARED((Np, F), F32),
            pltpu.VMEM((K, F), F32),
            pltpu.VMEM((K,), I32),
            pltpu.VMEM((K,), I32),
            pltpu.VMEM((K,), F32),
            pltpu.VMEM((ZB, F), F32),
        ],
        compiler_params=_sc_compiler_params(),
    )
    def k(z0_hbm, z1_hbm, rows_hbm, cols_hbm, vals_hbm, y0_hbm, y1_hbm,
          acc, gbuf, rowv, colv, valv, zbuf):
        cid = lax.axis_index("c")
        sid = lax.axis_index("s")

        zero = jnp.zeros((L,), F32)

        @pl.loop(0, ZB)
        def _(i):
            for j in range(F // L):
                zbuf[i, pl.ds(j * L, L)] = zero

        for j in range(RPT // ZB):
            zoff = pl.multiple_of(sid * RPT + j * ZB, 8)
            pltpu.sync_copy(zbuf, acc.at[pl.ds(zoff, ZB)])
        plsc.subcore_barrier()

        def run(z_hbm, y_hbm):
            @pl.loop(0, NCHUNK)
            def _(c):
                base = pl.multiple_of(sid * EPS + c * K, 8)
                pltpu.sync_copy(rows_hbm.at[pl.ds(base, K)], rowv)
                pltpu.sync_copy(cols_hbm.at[pl.ds(base, K)], colv)
                pltpu.sync_copy(vals_hbm.at[pl.ds(base, K)], valv)
                pltpu.sync_copy(z_hbm.at[colv], gbuf)

                @pl.loop(0, K)
                def _(i):
                    bval = plsc.load_gather(valv, [jnp.full((L,), i, I32)])
                    for j in range(F // L):
                        sl = pl.ds(j * L, L)
                        gbuf[i, sl] = gbuf[i, sl] * bval

                pltpu.sync_copy(gbuf, acc.at[rowv], add=True)

            plsc.subcore_barrier()
            woff = pl.multiple_of(sid * RPT, 8)
            pltpu.sync_copy(acc.at[pl.ds(woff, RPT)],
                            y_hbm.at[pl.ds(woff, RPT)])

        @pl.when(cid == 0)
        def _():
            run(z0_hbm, y0_hbm)

        @pl.when(cid == 1)
        def _():
            run(z1_hbm, y1_hbm)

    return k(z0, z1, rows, cols, vals)


def _k3_gather(y0, y1, xflat, idxc, idxx):
    F = y0.shape[1]
    Mp = idxc.shape[0]
    Xp = idxx.shape[0]
    CK = 80
    XK = 128
    mpt = Mp // NW
    xpt = Xp // NW
    mesh = plsc.VectorSubcoreMesh(core_axis_name="c", subcore_axis_name="s")

    @functools.partial(
        pl.kernel,
        out_type=(jax.ShapeDtypeStruct((Mp, F), F32),
                  jax.ShapeDtypeStruct((Mp, F), F32),
                  jax.ShapeDtypeStruct((Xp, F), F32)),
        mesh=mesh,
        scratch_types=[
            pltpu.VMEM((CK,), I32),
            pltpu.VMEM((CK, F), F32),
            pltpu.VMEM((XK,), I32),
            pltpu.VMEM((XK, F), F32),
        ],
    )
    def k(y0_hbm, y1_hbm, x_hbm, ic_hbm, ix_hbm,
          yc0_hbm, yc1_hbm, xc_hbm, icv, cbuf, ixv, xbuf):
        cid = lax.axis_index("c")
        sid = lax.axis_index("s")
        wid = sid * NC + cid

        @pl.loop(0, mpt // CK)
        def _(j):
            base = pl.multiple_of(wid * mpt + j * CK, 8)
            pltpu.sync_copy(ic_hbm.at[pl.ds(base, CK)], icv)
            pltpu.sync_copy(y0_hbm.at[icv], cbuf)
            pltpu.sync_copy(cbuf, yc0_hbm.at[pl.ds(base, CK)])
            pltpu.sync_copy(y1_hbm.at[icv], cbuf)
            pltpu.sync_copy(cbuf, yc1_hbm.at[pl.ds(base, CK)])

        @pl.loop(0, xpt // XK)
        def _(j):
            base = pl.multiple_of(wid * xpt + j * XK, 8)
            pltpu.sync_copy(ix_hbm.at[pl.ds(base, XK)], ixv)
            pltpu.sync_copy(x_hbm.at[ixv], xbuf)
            pltpu.sync_copy(xbuf, xc_hbm.at[pl.ds(base, XK)])

    return k(y0, y1, xflat, idxc, idxx)


def _k4_final(yc0, yc1, xc, W_eye, b_lin, b_eye, B, M, Mp):
    O = W_eye.shape[0]

    def body(yc0_ref, yc1_ref, xc_ref, w_ref, bl_ref, be_ref, o_ref):
        w = w_ref[...]
        bl = bl_ref[...][None, :]
        be = be_ref[...][None, :]
        for b in range(B):
            ycr = yc0_ref if b < 2 else yc1_ref
            linpart = ycr[pl.ds(0, M), pl.ds((b % 2) * O, O)]
            o_ref[b, :, 0:O] = jnp.maximum(linpart + bl, 0.0)
            xb = xc_ref[pl.ds(b * Mp, M), :]
            eye = lax.dot_general(
                xb, w, (((1,), (1,)), ((), ())),
                precision=lax.Precision.HIGHEST,
                preferred_element_type=F32)
            o_ref[b, :, O:2 * O] = jnp.maximum(eye + be, 0.0)

    return pl.pallas_call(
        body,
        out_shape=jax.ShapeDtypeStruct((B, M, 2 * O), F32),
    )(yc0, yc1, xc, W_eye, b_lin, b_eye)


def kernel(x, W_lin, b_lin, W_eye, b_eye, adj_vals, edge_index, centroids):
    B, N, CIN = x.shape
    O = W_lin.shape[0]
    E = adj_vals.shape[0]
    M = centroids.shape[0]

    EDGE_Q = NS * 128
    Ep = -(-E // EDGE_Q) * EDGE_Q
    rows = jnp.concatenate([edge_index[0], jnp.zeros((Ep - E,), I32)])
    cols = jnp.concatenate([edge_index[1], jnp.zeros((Ep - E,), I32)])
    vals = jnp.concatenate([adj_vals, jnp.zeros((Ep - E,), F32)])

    CEN_Q = NW * 80
    Mp = -(-M // CEN_Q) * CEN_Q
    idxc = jnp.concatenate([centroids, jnp.zeros((Mp - M,), I32)])
    cen_pad = jnp.concatenate([centroids, jnp.zeros((Mp - M,), I32)])
    idxx = (jnp.arange(B, dtype=I32)[:, None] * N + cen_pad[None, :]).reshape(-1)

    NODE_Q = NS * 8
    Np = -(-N // NODE_Q) * NODE_Q

    z0, z1 = _k1_linear(x, W_lin)
    y0, y1 = _k2_spmm(z0, z1, rows, cols, vals, Np)
    yc0, yc1, xc = _k3_gather(y0, y1, x.reshape(B * N, CIN), idxc, idxx)
    return _k4_final(yc0, yc1, xc, W_eye, b_lin, b_eye, B, M, Mp)


# speedup vs baseline: 4.5841x; 1.0327x over previous
import dataclasses
import functools

import jax
import jax.numpy as jnp
from jax import lax
from jax.experimental import pallas as pl
from jax.experimental.pallas import tpu as pltpu
from jax.experimental.pallas import tpu_sc as plsc

F32 = jnp.float32
I32 = jnp.int32

NC, NS, L = 2, 16, 16
NW = NC * NS


def _sc_compiler_params():
    cp = pltpu.CompilerParams()
    if "needs_layout_passes" in pltpu.CompilerParams.__dataclass_fields__:
        cp = dataclasses.replace(cp, needs_layout_passes=False)
    return cp


def _k1_linear(x, W_lin):
    B, N, CIN = x.shape
    O = W_lin.shape[0]
    RB = 2000

    def body(x_ref, w_ref, z_ref):
        w = w_ref[...]
        for h in range(2):
            for j in range(2):
                z_ref[h, :, j * O:(j + 1) * O] = lax.dot_general(
                    x_ref[h * 2 + j], w, (((1,), (1,)), ((), ())),
                    precision=lax.Precision.HIGHEST,
                    preferred_element_type=F32)

    return pl.pallas_call(
        body,
        grid=(N // RB,),
        in_specs=[
            pl.BlockSpec((B, RB, CIN), lambda i: (0, i, 0)),
            pl.BlockSpec((O, CIN), lambda i: (0, 0)),
        ],
        out_specs=pl.BlockSpec((2, RB, 2 * O), lambda i: (0, i, 0)),
        out_shape=jax.ShapeDtypeStruct((2, N, 2 * O), F32),
    )(x, W_lin)


def _k2_spmm_gather(z, rows, cols, vals, idxc, idxx, xflat, Np):
    F = z.shape[2]
    K = 128
    Ep = rows.shape[0]
    EPS = Ep // NS
    NCHUNK = EPS // K
    RPT = Np // NS
    ZB = 128
    Mp = idxc.shape[0]
    Xp = idxx.shape[0]
    CK = 80
    SPT = Mp // NS
    XK = 128
    XPT = Xp // NW
    mesh = plsc.VectorSubcoreMesh(core_axis_name="c", subcore_axis_name="s")

    @functools.partial(
        pl.kernel,
        out_type=(jax.ShapeDtypeStruct((2, Mp, F), F32),
                  jax.ShapeDtypeStruct((Xp, F), F32)),
        mesh=mesh,
        scratch_types=[
            pltpu.VMEM_SHARED((Np, F), F32),
            pltpu.VMEM((K, F), F32),
            pltpu.VMEM((K,), I32),
            pltpu.VMEM((K,), I32),
            pltpu.VMEM((K,), F32),
            pltpu.VMEM((CK,), I32),
            pltpu.VMEM((XK,), I32),
        ],
        compiler_params=_sc_compiler_params(),
    )
    def k(z_hbm, rows_hbm, cols_hbm, vals_hbm, ic_hbm, ix_hbm, x_hbm,
          yc_hbm, xc_hbm,
          acc, gbuf, rowv, colv, valv, icv, ixv):
        cid = lax.axis_index("c")
        sid = lax.axis_index("s")
        wid = sid * NC + cid

        zero = jnp.zeros((L,), F32)

        @pl.loop(0, ZB)
        def _(i):
            for j in range(F // L):
                gbuf[i, pl.ds(j * L, L)] = zero

        for j in range(RPT // ZB):
            zoff = pl.multiple_of(sid * RPT + j * ZB, 8)
            pltpu.sync_copy(gbuf, acc.at[pl.ds(zoff, ZB)])
        plsc.subcore_barrier()

        zsrc = z_hbm.at[cid]

        @pl.loop(0, NCHUNK)
        def _(c):
            base = pl.multiple_of(sid * EPS + c * K, 8)
            pltpu.sync_copy(rows_hbm.at[pl.ds(base, K)], rowv)
            pltpu.sync_copy(cols_hbm.at[pl.ds(base, K)], colv)
            pltpu.sync_copy(vals_hbm.at[pl.ds(base, K)], valv)
            pltpu.sync_copy(zsrc.at[colv], gbuf)

            @pl.loop(0, K)
            def _(i):
                bval = plsc.load_gather(valv, [jnp.full((L,), i, I32)])
                for j in range(F // L):
                    sl = pl.ds(j * L, L)
                    gbuf[i, sl] = gbuf[i, sl] * bval

            pltpu.sync_copy(gbuf, acc.at[rowv], add=True)

        plsc.subcore_barrier()

        ycdst = yc_hbm.at[cid]

        cbuf = gbuf.at[pl.ds(0, CK)]

        @pl.loop(0, SPT // CK)
        def _(j):
            base = pl.multiple_of(sid * SPT + j * CK, 8)
            pltpu.sync_copy(ic_hbm.at[pl.ds(base, CK)], icv)
            pltpu.sync_copy(acc.at[icv], cbuf)
            pltpu.sync_copy(cbuf, ycdst.at[pl.ds(base, CK)])

        @pl.loop(0, XPT // XK)
        def _(j):
            base = pl.multiple_of(wid * XPT + j * XK, 8)
            pltpu.sync_copy(ix_hbm.at[pl.ds(base, XK)], ixv)
            pltpu.sync_copy(x_hbm.at[ixv], gbuf)
            pltpu.sync_copy(gbuf, xc_hbm.at[pl.ds(base, XK)])

    return k(z, rows, cols, vals, idxc, idxx, xflat)


def _k3_final(yc, xc, W_eye, b_lin, b_eye, B, M, Mp):
    O = W_eye.shape[0]

    def body(yc_ref, xc_ref, w_ref, bl_ref, be_ref, o_ref):
        w = w_ref[...]
        bl = bl_ref[...][None, :]
        be = be_ref[...][None, :]
        for b in range(B):
            linpart = yc_ref[b // 2, pl.ds(0, M), pl.ds((b % 2) * O, O)]
            o_ref[b, :, 0:O] = jnp.maximum(linpart + bl, 0.0)
            xb = xc_ref[pl.ds(b * Mp, M), :]
            eye = lax.dot_general(
                xb, w, (((1,), (1,)), ((), ())),
                precision=lax.Precision.HIGHEST,
                preferred_element_type=F32)
            o_ref[b, :, O:2 * O] = jnp.maximum(eye + be, 0.0)

    return pl.pallas_call(
        body,
        out_shape=jax.ShapeDtypeStruct((B, M, 2 * O), F32),
    )(yc, xc, W_eye, b_lin, b_eye)


def kernel(x, W_lin, b_lin, W_eye, b_eye, adj_vals, edge_index, centroids):
    B, N, CIN = x.shape
    E = adj_vals.shape[0]
    M = centroids.shape[0]

    EDGE_Q = NS * 128
    Ep = -(-E // EDGE_Q) * EDGE_Q
    rows = jnp.concatenate([edge_index[0], jnp.zeros((Ep - E,), I32)])
    cols = jnp.concatenate([edge_index[1], jnp.zeros((Ep - E,), I32)])
    vals = jnp.concatenate([adj_vals, jnp.zeros((Ep - E,), F32)])

    CEN_Q = NS * 80
    Mp = -(-M // CEN_Q) * CEN_Q
    idxc = jnp.concatenate([centroids, jnp.zeros((Mp - M,), I32)])
    idxx = (jnp.arange(B, dtype=I32)[:, None] * N + idxc[None, :]).reshape(-1)

    NODE_Q = NS * 8
    Np = -(-N // NODE_Q) * NODE_Q

    z = _k1_linear(x, W_lin)
    yc, xc = _k2_spmm_gather(z, rows, cols, vals, idxc, idxx,
                             x.reshape(B * N, CIN), Np)
    return _k3_final(yc, xc, W_eye, b_lin, b_eye, B, M, Mp)
